# baseline (speedup 1.0000x reference)
import functools

import jax
import jax.numpy as jnp
from jax import lax
from jax.experimental import pallas as pl
from jax.experimental.pallas import tpu as pltpu
from jax.experimental.pallas import tpu_sc as plsc

B = 16384
L = 50
LG = 56
D = 64
BT = B * L
NC = 2
NS = 16
NW = NC * NS
BPW = B // NW
NBC = 16
NCH = BPW // NBC

_mesh = plsc.VectorSubcoreMesh(core_axis_name="c", subcore_axis_name="s")


@functools.partial(
    pl.kernel,
    mesh=_mesh,
    out_type=jax.ShapeDtypeStruct((BT, D), jnp.float32),
    scratch_types=[
        pltpu.VMEM((NBC, 128), jnp.int32),
        pltpu.VMEM((NBC, LG, D), jnp.float32),
        pltpu.SemaphoreType.DMA,
    ],
    compiler_params=pltpu.CompilerParams(use_tc_tiling_on_sc=False),
)
def _gather_kernel(idx_hbm, table_hbm, out_hbm, idx_v, rows_v, sem):
    wid = lax.axis_index("s") * NC + lax.axis_index("c")
    b_base = wid * BPW

    def body(g, carry):
        bb = b_base + g * NBC
        pltpu.sync_copy(idx_hbm.at[pl.ds(bb, NBC)], idx_v)
        descs = [
            pltpu.async_copy(
                table_hbm.at[idx_v.at[b, pl.ds(0, LG)]], rows_v.at[b], sem)
            for b in range(NBC)
        ]
        for d in descs:
            d.wait()
        for b in range(NBC):
            pltpu.sync_copy(rows_v.at[b, pl.ds(0, L)],
                            out_hbm.at[pl.ds((bb + b) * L, L)])
        return carry

    lax.fori_loop(0, NCH, body, 0)


def kernel(inputs, word_table):
    fill = (jnp.arange(B, dtype=jnp.int32)[:, None] * 8
            + jnp.arange(128 - L, dtype=jnp.int32)[None, :]) % (2 ** 20)
    idx_pad = jnp.concatenate([inputs.astype(jnp.int32), fill], axis=1)
    idx_pad = jax.lax.optimization_barrier(idx_pad)
    out = _gather_kernel(idx_pad, word_table)
    out = jax.lax.optimization_barrier(out)
    return out.reshape(B, L, D)


# speedup vs baseline: 1.3316x; 1.3316x over previous
import functools

import jax
import jax.numpy as jnp
from jax import lax
from jax.experimental import pallas as pl
from jax.experimental.pallas import tpu as pltpu
from jax.experimental.pallas import tpu_sc as plsc

B = 16384
L = 50
D = 64
NC = 2
NS = 16
NW = NC * NS
BPW = B // NW
NBC = 16
NCH = BPW // NBC

_mesh = plsc.VectorSubcoreMesh(core_axis_name="c", subcore_axis_name="s")


@functools.partial(
    pl.kernel,
    mesh=_mesh,
    out_type=jax.ShapeDtypeStruct((B, L, D), jnp.float32),
    scratch_types=[
        pltpu.VMEM((NBC, L), jnp.int32),
        pltpu.VMEM((NBC, L, D), jnp.float32),
        pltpu.SemaphoreType.DMA,
    ],
    compiler_params=pltpu.CompilerParams(use_tc_tiling_on_sc=False),
)
def _gather_kernel(idx_hbm, table_hbm, out_hbm, idx_v, rows_v, sem):
    wid = lax.axis_index("s") * NC + lax.axis_index("c")
    b_base = wid * BPW

    def body(g, carry):
        bb = b_base + g * NBC
        pltpu.sync_copy(idx_hbm.at[pl.ds(bb, NBC)], idx_v)
        descs = [
            pltpu.async_copy(table_hbm.at[idx_v.at[b]], rows_v.at[b], sem)
            for b in range(NBC)
        ]
        for d in descs:
            d.wait()
        pltpu.sync_copy(rows_v, out_hbm.at[pl.ds(bb, NBC)])
        return carry

    lax.fori_loop(0, NCH, body, 0)


def kernel(inputs, word_table):
    return _gather_kernel(inputs.astype(jnp.int32), word_table)


# speedup vs baseline: 1.3553x; 1.0178x over previous
import functools

import jax
import jax.numpy as jnp
from jax import lax
from jax.experimental import pallas as pl
from jax.experimental.pallas import tpu as pltpu
from jax.experimental.pallas import tpu_sc as plsc

B = 16384
L = 50
D = 64
NC = 2
NS = 16
NW = NC * NS
BPW = B // NW
NBC = 32
NCH = BPW // NBC

_mesh = plsc.VectorSubcoreMesh(core_axis_name="c", subcore_axis_name="s")


@functools.partial(
    pl.kernel,
    mesh=_mesh,
    out_type=jax.ShapeDtypeStruct((B, L, D), jnp.float32),
    scratch_types=[
        pltpu.VMEM((NBC, L), jnp.int32),
        pltpu.VMEM((NBC, L, D), jnp.float32),
        pltpu.SemaphoreType.DMA,
    ],
    compiler_params=pltpu.CompilerParams(use_tc_tiling_on_sc=False),
)
def _gather_kernel(idx_hbm, table_hbm, out_hbm, idx_v, rows_v, sem):
    wid = lax.axis_index("s") * NC + lax.axis_index("c")
    b_base = wid * BPW

    def body(g, carry):
        bb = b_base + g * NBC
        pltpu.sync_copy(idx_hbm.at[pl.ds(bb, NBC)], idx_v)
        descs = [
            pltpu.async_copy(table_hbm.at[idx_v.at[b]], rows_v.at[b], sem)
            for b in range(NBC)
        ]
        for d in descs:
            d.wait()
        pltpu.sync_copy(rows_v, out_hbm.at[pl.ds(bb, NBC)])
        return carry

    lax.fori_loop(0, NCH, body, 0)


def kernel(inputs, word_table):
    return _gather_kernel(inputs.astype(jnp.int32), word_table)
